# Initial kernel scaffold; baseline (speedup 1.0000x reference)
import jax
import jax.numpy as jnp
from jax.experimental import pallas as pl


def kernel(features, coarse_logits, W1, b1, W2, b2, W3, b3, Wp, bp):
    raise NotImplementedError("write your pallas kernel here")


# speedup vs baseline: 2.1335x; 2.1335x over previous
import functools

import jax
import jax.numpy as jnp
import numpy as np
from jax import lax
from jax.experimental import pallas as pl
from jax.experimental.pallas import tpu as pltpu
from jax.experimental.pallas import tpu_sc as plsc

NCLS = 19
_INTERP = False
NPT = 8192
FDIM = 256
IDXPAD = NPT + 256


def _upmat(h):
  o = np.arange(2 * h)
  coord = o * 0.5 - 0.25
  i0 = np.floor(coord).astype(np.int64)
  w1 = (coord - i0).astype(np.float32)
  u = np.zeros((h, 2 * h), np.float32)
  np.add.at(u, (np.clip(i0, 0, h - 1), o), 1.0 - w1)
  np.add.at(u, (np.clip(i0 + 1, 0, h - 1), o), w1)
  return jnp.asarray(u)


def _k1a_body(x_ref, uw_ref, o_ref):
  x = x_ref[0, 0]
  o_ref[0, 0] = jnp.dot(x, uw_ref[...], preferred_element_type=jnp.float32,
                        precision=lax.Precision.HIGHEST)


def _k1a(sem, uw):
  n, c, h, w = sem.shape
  return pl.pallas_call(
      _k1a_body,
      interpret=_INTERP,
      grid=(n, c),
      in_specs=[
          pl.BlockSpec((1, 1, h, w), lambda i, j: (i, j, 0, 0)),
          pl.BlockSpec((w, 2 * w), lambda i, j: (0, 0)),
      ],
      out_specs=pl.BlockSpec((1, 1, h, 2 * w), lambda i, j: (i, j, 0, 0)),
      out_shape=jax.ShapeDtypeStruct((n, c, h, 2 * w), jnp.float32),
  )(sem, uw)


def _k1b_body(x_ref, uh_ref, up_ref, key_ref):
  uh = uh_ref[...]
  m1 = None
  m2 = None
  for c in range(NCLS):
    u = jnp.dot(uh, x_ref[0, c], preferred_element_type=jnp.float32,
                precision=lax.Precision.HIGHEST)
    up_ref[0, c] = u
    if c == 0:
      m1 = u
      m2 = jnp.full_like(u, -jnp.inf)
    else:
      nm1 = jnp.maximum(m1, u)
      m2 = jnp.maximum(m2, jnp.minimum(m1, u))
      m1 = nm1
  unc = m2 - m1
  b = lax.bitcast_convert_type(unc, jnp.int32)
  key_ref[0] = jnp.where(b < 0, b ^ jnp.int32(0x7FFFFFFF), b)


def _k1b(xw, uh):
  n, c, h, w2 = xw.shape
  h2 = 2 * h
  wt = 128
  nw = w2 // wt
  return pl.pallas_call(
      _k1b_body,
      interpret=_INTERP,
      grid=(n, nw),
      in_specs=[
          pl.BlockSpec((1, c, h, wt), lambda i, j: (i, 0, 0, j)),
          pl.BlockSpec((h2, h), lambda i, j: (0, 0)),
      ],
      out_specs=[
          pl.BlockSpec((1, c, h2, wt), lambda i, j: (i, 0, 0, j)),
          pl.BlockSpec((1, h2, wt), lambda i, j: (i, 0, j)),
      ],
      out_shape=[
          jax.ShapeDtypeStruct((n, c, h2, w2), jnp.float32),
          jax.ShapeDtypeStruct((n, h2, w2), jnp.int32),
      ],
  )(xw, uh)


def _make_topk(hw):
  ch = hw // 16
  nsl = ch // 16
  och = IDXPAD // 16
  mesh = plsc.VectorSubcoreMesh(core_axis_name="c", subcore_axis_name="s", num_cores=2, num_subcores=16)

  @functools.partial(
      pl.kernel,
      out_type=jax.ShapeDtypeStruct((2 * IDXPAD,), jnp.int32),
      mesh=mesh,
      compiler_params=pltpu.CompilerParams(needs_layout_passes=False),
      interpret=_INTERP,
      scratch_types=[
          pltpu.VMEM((ch,), jnp.int32),
          pltpu.VMEM((16,), jnp.int32),
          pltpu.VMEM((256,), jnp.int32),
          pltpu.VMEM((IDXPAD,), jnp.int32),
          pltpu.VMEM((och,), jnp.int32),
          pltpu.VMEM((och,), jnp.int32),
          pltpu.VMEM_SHARED((256,), jnp.int32),
          pltpu.VMEM_SHARED((16 * IDXPAD,), jnp.int32),
      ],
  )
  def k(keys_hbm, idx_hbm, keys_v, cnt_v, all_v, buf_v, acc_v, tmp_v, shc, shb):
    n = lax.axis_index("c")
    sid = lax.axis_index("s")
    sid16 = jnp.full((16,), sid, jnp.int32)
    base = sid * ch
    pltpu.sync_copy(keys_hbm.at[pl.ds(n * (16 * ch) + base, ch)], keys_v)

    kvec = jnp.full((16,), NPT, jnp.int32)
    zero = jnp.zeros((16,), jnp.int32)
    one = jnp.full((16,), 1, jnp.int32)

    def count_ge(tv):
      def body(i, acc):
        v = keys_v[pl.ds(i * 16, 16)]
        return acc + jnp.where(v >= tv, one, zero)
      acc = lax.fori_loop(0, nsl, body, zero)
      return jnp.sum(acc)

    def exchange(local_scalar):
      cnt_v[...] = jnp.full((16,), local_scalar, jnp.int32)
      pltpu.sync_copy(cnt_v, shc.at[pl.ds(sid * 16, 16)])
      plsc.subcore_barrier()
      pltpu.sync_copy(shc, all_v)
      plsc.subcore_barrier()
      tot = zero
      pre = zero
      for j in range(16):
        row = all_v[pl.ds(j * 16, 16)]
        tot = tot + row
        pre = pre + jnp.where(jnp.full((16,), j, jnp.int32) < sid16, row, zero)
      return tot, pre

    def sbody(_, carry):
      lo, hi = carry
      mid = (lo >> 1) + (hi >> 1) + (lo & hi & 1)
      tot, _ = exchange(count_ge(mid))
      pred = tot >= kvec
      return (jnp.where(pred, mid, lo), jnp.where(pred, hi, mid))

    lo0 = jnp.full((16,), jnp.int32(-(2**31)), jnp.int32)
    hi0 = jnp.full((16,), jnp.int32(2**31 - 1), jnp.int32)
    lo, _ = lax.fori_loop(0, 32, sbody, (lo0, hi0))
    vstar = lo

    def cbody(i, carry):
      ag, ae = carry
      v = keys_v[pl.ds(i * 16, 16)]
      ag = ag + jnp.where(v > vstar, one, zero)
      ae = ae + jnp.where(v == vstar, one, zero)
      return (ag, ae)

    ag, ae = lax.fori_loop(0, nsl, cbody, (zero, zero))
    tot_g, pre_g = exchange(jnp.sum(ag))
    _, pre_e = exchange(jnp.sum(ae))
    mvec = kvec - tot_g

    def zbody(i, _):
      buf_v[pl.ds(i * 16, 16)] = zero
      return 0

    lax.fori_loop(0, IDXPAD // 16, zbody, 0)

    gbase = jnp.full((16,), base, jnp.int32) + lax.iota(jnp.int32, 16)

    def pbody(i, carry):
      rg, re = carry
      v = keys_v[pl.ds(i * 16, 16)]
      mg = v > vstar
      me = v == vstar
      cg = plsc.cumsum(jnp.where(mg, one, zero))
      ce = plsc.cumsum(jnp.where(me, one, zero))
      posg = pre_g + rg + cg - one
      grank = pre_e + re + ce - one
      take = me & (grank < mvec)
      pose = tot_g + grank
      pos = jnp.where(mg, posg, pose)
      sel = mg | take
      gidx = gbase + jnp.full((16,), i * 16, jnp.int32)
      plsc.store_scatter(buf_v, [pos], gidx, mask=sel)
      return (rg + cg[15], re + ce[15])

    lax.fori_loop(0, nsl, pbody, (zero, zero))

    pltpu.sync_copy(buf_v, shb.at[pl.ds(sid * IDXPAD, IDXPAD)])
    plsc.subcore_barrier()
    obase = sid * och

    def z2body(i, _):
      acc_v[pl.ds(i * 16, 16)] = zero
      return 0

    lax.fori_loop(0, och // 16, z2body, 0)
    for t in range(16):
      pltpu.sync_copy(shb.at[pl.ds(t * IDXPAD + obase, och)], tmp_v)
      def abody(i, _):
        acc_v[pl.ds(i * 16, 16)] = (
            acc_v[pl.ds(i * 16, 16)] + tmp_v[pl.ds(i * 16, 16)])
        return 0
      lax.fori_loop(0, och // 16, abody, 0)
    pltpu.sync_copy(acc_v, idx_hbm.at[pl.ds(n * IDXPAD + obase, och)])
    plsc.subcore_barrier()

  return k


def _make_gather(hw_side, shift, wtab):
  ppt = NPT // 16
  nch = ppt // 128
  mesh = plsc.VectorSubcoreMesh(core_axis_name="c", subcore_axis_name="s", num_cores=2, num_subcores=16)
  w1tab = [jnp.float32(x) for x in wtab]
  mask_lo = (1 << shift) - 1

  @functools.partial(
      pl.kernel,
      out_type=jax.ShapeDtypeStruct((2, NPT, 128), jnp.float32),
      mesh=mesh,
      compiler_params=pltpu.CompilerParams(needs_layout_passes=False),
      interpret=_INTERP,
      scratch_types=[
          pltpu.VMEM((ppt,), jnp.int32),
          pltpu.VMEM((4, 128), jnp.int32),
          pltpu.VMEM((128, 16), jnp.float32),
          pltpu.VMEM((4, 128, 128), jnp.float32),
          pltpu.VMEM((128, 128), jnp.float32),
          pltpu.SemaphoreType.DMA,
      ],
  )
  def k(idx_hbm, tab_hbm, pts_hbm, idx_v, ti_v, tw_v, rows_v, out_v, sem):
    n = lax.axis_index("c")
    sid = lax.axis_index("s")
    pbase = sid * ppt
    pltpu.sync_copy(idx_hbm.at[pl.ds(n * IDXPAD + pbase, ppt)], idx_v)

    zero = jnp.zeros((16,), jnp.float32)

    def w1_of(r):
      w = jnp.full((16,), w1tab[0], jnp.float32)
      for j in range(1, len(w1tab)):
        w = jnp.where(r == j, jnp.full((16,), w1tab[j], jnp.float32), w)
      return w

    for chnk in range(nch):
      for sl in range(8):
        pix = idx_v[pl.ds(chnk * 128 + sl * 16, 16)]
        ix = pix & jnp.int32(hw_side - 1)
        iy = pix >> jnp.int32(hw_side.bit_length() - 1)
        x0 = (ix - jnp.int32(1 << (shift - 1))) >> jnp.int32(shift)
        y0 = (iy - jnp.int32(1 << (shift - 1))) >> jnp.int32(shift)
        wx1 = w1_of(ix & jnp.int32(mask_lo))
        wy1 = w1_of(iy & jnp.int32(mask_lo))
        wx0 = 1.0 - wx1
        wy0 = 1.0 - wy1
        x1 = x0 + 1
        y1 = y0 + 1
        vx0 = x0 >= 0
        vx1 = x1 <= 127
        vy0 = y0 >= 0
        vy1 = y1 <= 127
        xc0 = jnp.maximum(x0, 0)
        xc1 = jnp.minimum(x1, 127)
        yc0 = jnp.maximum(y0, 0)
        yc1 = jnp.minimum(y1, 127)
        taps = [
            (yc0, xc0, jnp.where(vy0 & vx0, wy0 * wx0, zero)),
            (yc0, xc1, jnp.where(vy0 & vx1, wy0 * wx1, zero)),
            (yc1, xc0, jnp.where(vy1 & vx0, wy1 * wx0, zero)),
            (yc1, xc1, jnp.where(vy1 & vx1, wy1 * wx1, zero)),
        ]
        rowi = jnp.full((16,), sl * 16, jnp.int32) + lax.iota(jnp.int32, 16)
        for t, (yy, xx, ww) in enumerate(taps):
          ti_v[t, pl.ds(sl * 16, 16)] = yy * 128 + xx
          plsc.store_scatter(
              tw_v, [rowi, jnp.full((16,), t, jnp.int32)], ww)

      cps = [
          pltpu.async_copy(tab_hbm.at[n].at[ti_v.at[t]], rows_v.at[t], sem)
          for t in range(4)
      ]
      for cp in cps:
        cp.wait()

      def comb(p, _):
        wrow = tw_v[p, :]
        w0 = jnp.full((16,), wrow[0], jnp.float32)
        w1 = jnp.full((16,), wrow[1], jnp.float32)
        w2 = jnp.full((16,), wrow[2], jnp.float32)
        w3 = jnp.full((16,), wrow[3], jnp.float32)
        for kk in range(8):
          s = pl.ds(kk * 16, 16)
          acc = (w0 * rows_v[0, p, s] + w1 * rows_v[1, p, s]
                 + w2 * rows_v[2, p, s] + w3 * rows_v[3, p, s])
          out_v[p, s] = acc
        return 0

      lax.fori_loop(0, 128, comb, 0)
      pltpu.sync_copy(out_v, pts_hbm.at[n, pl.ds(pbase + chnk * 128, 128), :])

  return k


def _k4_body(x_ref, w1_ref, w2h_ref, w2c_ref, w3h_ref, w3c_ref, wph_ref,
             wpc_ref, b1_ref, b2_ref, b3_ref, bp_ref, o_ref):
  bf = jnp.bfloat16

  def dot(a, w):
    return jnp.dot(a.astype(bf), w.astype(bf),
                   preferred_element_type=jnp.float32)

  x0 = x_ref[0]
  h = jnp.maximum(dot(x0, w1_ref[...]) + b1_ref[0:1, :], 0.0)
  h = jnp.maximum(dot(h, w2h_ref[...]) + dot(x0, w2c_ref[...])
                  + b2_ref[0:1, :], 0.0)
  h = jnp.maximum(dot(h, w3h_ref[...]) + dot(x0, w3c_ref[...])
                  + b3_ref[0:1, :], 0.0)
  o = (dot(h, wph_ref[...]) + dot(x0, wpc_ref[...]) + bp_ref[0:1, :])
  o_ref[0] = o


def _k4(pts, wd):
  n = pts.shape[0]
  pb = 2048
  npb = NPT // pb
  wspecs = [pl.BlockSpec(w.shape, lambda i, j: tuple([0] * w.ndim))
            for w in wd]
  return pl.pallas_call(
      _k4_body,
      interpret=_INTERP,
      grid=(n, npb),
      in_specs=[pl.BlockSpec((1, pb, 128), lambda i, j: (i, j, 0))] + wspecs,
      out_specs=pl.BlockSpec((1, pb, 32), lambda i, j: (i, j, 0)),
      out_shape=jax.ShapeDtypeStruct((n, NPT, 32), jnp.float32),
  )(pts, *wd)


def _make_scatter(hw):
  cpix = hw // 16
  ppt = NPT // 16
  mesh = plsc.VectorSubcoreMesh(core_axis_name="c", subcore_axis_name="s", num_cores=2, num_subcores=16)

  @functools.partial(
      pl.kernel,
      out_type=jax.ShapeDtypeStruct((2 * NCLS * hw,), jnp.float32),
      mesh=mesh,
      compiler_params=pltpu.CompilerParams(needs_layout_passes=False),
      interpret=_INTERP,
      scratch_types=[
          pltpu.VMEM((ppt,), jnp.int32),
          pltpu.VMEM((4, 128), jnp.int32),
          pltpu.VMEM((ppt,), jnp.float32),
          pltpu.VMEM((4, 128), jnp.float32),
          pltpu.SemaphoreType.DMA,
      ],
  )
  def k(up_hbm, idx_hbm, plt_hbm, out_hbm, idx_v, idxc_v, valf_v, val_v, sem):
    n = lax.axis_index("c")
    sid = lax.axis_index("s")
    for c in range(NCLS):
      cb = (n * NCLS + c) * hw + sid * cpix
      pltpu.sync_copy(up_hbm.at[pl.ds(cb, cpix)], out_hbm.at[pl.ds(cb, cpix)])
    pltpu.sync_copy(idx_hbm.at[pl.ds(n * IDXPAD + sid * ppt, ppt)], idx_v)
    plsc.subcore_barrier()
    for c in range(NCLS):
      pltpu.sync_copy(
          plt_hbm.at[pl.ds((n * 32 + c) * NPT + sid * ppt, ppt)], valf_v)
      cbase = jnp.full((16,), (n * NCLS + c) * hw, jnp.int32)
      for j in range(4):
        for kk in range(8):
          s = pl.ds(j * 128 + kk * 16, 16)
          s2 = pl.ds(kk * 16, 16)
          idxc_v[j, s2] = idx_v[s] + cbase
          val_v[j, s2] = valf_v[s]
      cps = [
          pltpu.async_copy(val_v.at[j], out_hbm.at[idxc_v.at[j]], sem)
          for j in range(4)
      ]
      for cp in cps:
        cp.wait()

  return k


def kernel(features, coarse_logits, W1, b1, W2, b2, W3, b3, Wp, bp):
  n, f, h0, w0 = features.shape
  c = coarse_logits.shape[1]

  tab = jnp.concatenate([features, coarse_logits], axis=1)
  tab = tab.reshape(n, f + c, h0 * w0).transpose(0, 2, 1)
  tab = jnp.pad(tab, ((0, 0), (0, 0), (0, 128 - (f + c))))

  w1t = jnp.zeros((128, FDIM), jnp.float32).at[: f + c].set(W1.T)
  w2h = W2[:, :FDIM].T
  w2c = jnp.zeros((128, FDIM), jnp.float32).at[f : f + c].set(W2[:, FDIM:].T)
  w3h = W3[:, :FDIM].T
  w3c = jnp.zeros((128, FDIM), jnp.float32).at[f : f + c].set(W3[:, FDIM:].T)
  wph = jnp.zeros((FDIM, 32), jnp.float32).at[:, :c].set(Wp[:, :FDIM].T)
  wpc = jnp.zeros((128, 32), jnp.float32).at[f : f + c, :c].set(Wp[:, FDIM:].T)
  b1r = jnp.broadcast_to(b1, (8, FDIM))
  b2r = jnp.broadcast_to(b2, (8, FDIM))
  b3r = jnp.broadcast_to(b3, (8, FDIM))
  bpr = jnp.zeros((8, 32), jnp.float32).at[0, :c].set(bp)
  wd = [w1t, w2h, w2c, w3h, w3c, wph, wpc, b1r, b2r, b3r, bpr]

  sem = coarse_logits
  for step in range(2):
    hin = sem.shape[2]
    h2 = 2 * hin
    hw = h2 * h2
    uw = _upmat(hin)
    uh = uw.T
    xw = _k1a(sem, uw)
    up, key = _k1b(xw, uh)
    idx = _make_topk(hw)(key.reshape(n * hw))
    shift = 1 if step == 0 else 2
    wtab = (0.75, 0.25) if step == 0 else (0.625, 0.875, 0.125, 0.375)
    pts = _make_gather(h2, shift, wtab)(idx, tab)
    po = _k4(pts, wd)
    plt = jnp.swapaxes(po, 1, 2).reshape(n * 32 * NPT)
    out = _make_scatter(hw)(up.reshape(n * c * hw), idx, plt)
    sem = out.reshape(n, c, h2, h2)
  return sem


# speedup vs baseline: 2.1716x; 1.0178x over previous
import functools

import jax
import jax.numpy as jnp
import numpy as np
from jax import lax
from jax.experimental import pallas as pl
from jax.experimental.pallas import tpu as pltpu
from jax.experimental.pallas import tpu_sc as plsc

NCLS = 19
_INTERP = False
NPT = 8192
FDIM = 256
IDXPAD = NPT + 256


def _upmat(h):
  o = np.arange(2 * h)
  coord = o * 0.5 - 0.25
  i0 = np.floor(coord).astype(np.int64)
  w1 = (coord - i0).astype(np.float32)
  u = np.zeros((h, 2 * h), np.float32)
  np.add.at(u, (np.clip(i0, 0, h - 1), o), 1.0 - w1)
  np.add.at(u, (np.clip(i0 + 1, 0, h - 1), o), w1)
  return jnp.asarray(u)


def _k1a_body(x_ref, uw_ref, o_ref):
  x = x_ref[0, 0]
  o_ref[0, 0] = jnp.dot(x, uw_ref[...], preferred_element_type=jnp.float32,
                        precision=lax.Precision.HIGHEST)


def _k1a(sem, uw):
  n, c, h, w = sem.shape
  return pl.pallas_call(
      _k1a_body,
      interpret=_INTERP,
      grid=(n, c),
      in_specs=[
          pl.BlockSpec((1, 1, h, w), lambda i, j: (i, j, 0, 0)),
          pl.BlockSpec((w, 2 * w), lambda i, j: (0, 0)),
      ],
      out_specs=pl.BlockSpec((1, 1, h, 2 * w), lambda i, j: (i, j, 0, 0)),
      out_shape=jax.ShapeDtypeStruct((n, c, h, 2 * w), jnp.float32),
  )(sem, uw)


def _k1b_body(x_ref, uh_ref, up_ref, key_ref):
  uh = uh_ref[...]
  m1 = None
  m2 = None
  for c in range(NCLS):
    u = jnp.dot(uh, x_ref[0, c], preferred_element_type=jnp.float32,
                precision=lax.Precision.HIGHEST)
    up_ref[0, c] = u
    if c == 0:
      m1 = u
      m2 = jnp.full_like(u, -jnp.inf)
    else:
      nm1 = jnp.maximum(m1, u)
      m2 = jnp.maximum(m2, jnp.minimum(m1, u))
      m1 = nm1
  unc = m2 - m1
  b = lax.bitcast_convert_type(unc, jnp.int32)
  key_ref[0] = jnp.where(b < 0, b ^ jnp.int32(0x7FFFFFFF), b)


def _k1b(xw, uh):
  n, c, h, w2 = xw.shape
  h2 = 2 * h
  wt = 128
  nw = w2 // wt
  return pl.pallas_call(
      _k1b_body,
      interpret=_INTERP,
      grid=(n, nw),
      in_specs=[
          pl.BlockSpec((1, c, h, wt), lambda i, j: (i, 0, 0, j)),
          pl.BlockSpec((h2, h), lambda i, j: (0, 0)),
      ],
      out_specs=[
          pl.BlockSpec((1, c, h2, wt), lambda i, j: (i, 0, 0, j)),
          pl.BlockSpec((1, h2, wt), lambda i, j: (i, 0, j)),
      ],
      out_shape=[
          jax.ShapeDtypeStruct((n, c, h2, w2), jnp.float32),
          jax.ShapeDtypeStruct((n, h2, w2), jnp.int32),
      ],
  )(xw, uh)


def _make_topk(hw):
  ch = hw // 16
  nsl = ch // 16
  och = IDXPAD // 16
  mesh = plsc.VectorSubcoreMesh(core_axis_name="c", subcore_axis_name="s", num_cores=2, num_subcores=16)

  @functools.partial(
      pl.kernel,
      out_type=jax.ShapeDtypeStruct((2 * IDXPAD,), jnp.int32),
      mesh=mesh,
      compiler_params=pltpu.CompilerParams(needs_layout_passes=False),
      interpret=_INTERP,
      scratch_types=[
          pltpu.VMEM((ch,), jnp.int32),
          pltpu.VMEM((16,), jnp.int32),
          pltpu.VMEM((256,), jnp.int32),
          pltpu.VMEM((IDXPAD,), jnp.int32),
          pltpu.VMEM((och,), jnp.int32),
          pltpu.VMEM((och,), jnp.int32),
          pltpu.VMEM_SHARED((768,), jnp.int32),
          pltpu.VMEM_SHARED((16 * IDXPAD,), jnp.int32),
      ],
  )
  def k(keys_hbm, idx_hbm, keys_v, cnt_v, all_v, buf_v, acc_v, tmp_v, shc, shb):
    n = lax.axis_index("c")
    sid = lax.axis_index("s")
    sid16 = jnp.full((16,), sid, jnp.int32)
    base = sid * ch
    pltpu.sync_copy(keys_hbm.at[pl.ds(n * (16 * ch) + base, ch)], keys_v)

    kvec = jnp.full((16,), NPT, jnp.int32)
    zero = jnp.zeros((16,), jnp.int32)
    one = jnp.full((16,), 1, jnp.int32)
    lanes = lax.iota(jnp.int32, 16)
    u32 = jnp.uint32

    def tbody(i, _):
      s = pl.ds(i * 16, 16)
      v = lax.bitcast_convert_type(keys_v[s], u32) ^ u32(0x80000000)
      keys_v[s] = lax.bitcast_convert_type(v, jnp.int32)
      return 0

    lax.fori_loop(0, nsl, tbody, 0)

    def uload(s):
      return lax.bitcast_convert_type(keys_v[s], u32)

    def exchange(cnt_vec, slot):
      sb = (slot % 3) * 256
      cnt_v[...] = cnt_vec
      pltpu.sync_copy(cnt_v, shc.at[pl.ds(sb + sid * 16, 16)])
      plsc.subcore_barrier()
      pltpu.sync_copy(shc.at[pl.ds(sb, 256)], all_v)
      tot = zero
      pre = zero
      for j in range(16):
        row = all_v[pl.ds(j * 16, 16)]
        tot = tot + row
        pre = pre + jnp.where(jnp.full((16,), j, jnp.int32) < sid16, row, zero)
      return tot, pre

    p = jnp.full((16,), 0, u32)
    for r in range(8):
      shift = 28 - 4 * r
      ts = [p + u32(j << shift) for j in range(1, 16)]

      def rbody(i, accs):
        uv = uload(pl.ds(i * 16, 16))
        return tuple(a + jnp.where(uv >= t, one, zero)
                     for a, t in zip(accs, ts))

      accs = lax.fori_loop(0, nsl, rbody, (zero,) * 15)
      cvec = zero
      for j in range(15):
        cvec = jnp.where(lanes == j + 1,
                         jnp.full((16,), jnp.sum(accs[j]), jnp.int32), cvec)
      tot, _ = exchange(cvec, r)
      s = jnp.sum(jnp.where(tot >= kvec, one, zero))
      p = p + lax.convert_element_type(
          jnp.full((16,), lax.shift_left(s, shift), jnp.int32), u32)

    ustar = p

    def cbody(i, carry):
      ag, ae = carry
      uv = uload(pl.ds(i * 16, 16))
      ag = ag + jnp.where(uv > ustar, one, zero)
      ae = ae + jnp.where(uv == ustar, one, zero)
      return (ag, ae)

    ag, ae = lax.fori_loop(0, nsl, cbody, (zero, zero))
    c2 = jnp.where(lanes == 0, jnp.full((16,), jnp.sum(ag), jnp.int32),
                   jnp.where(lanes == 1,
                             jnp.full((16,), jnp.sum(ae), jnp.int32), zero))
    tot2, pre2 = exchange(c2, 8)
    tot_g = jnp.full((16,), tot2[0], jnp.int32)
    pre_g = jnp.full((16,), pre2[0], jnp.int32)
    pre_e = jnp.full((16,), pre2[1], jnp.int32)
    mvec = kvec - tot_g

    def zbody(i, _):
      buf_v[pl.ds(i * 16, 16)] = zero
      return 0

    lax.fori_loop(0, IDXPAD // 16, zbody, 0)

    gbase = jnp.full((16,), base, jnp.int32) + lax.iota(jnp.int32, 16)

    def pbody(i, carry):
      rg, re = carry
      uv = uload(pl.ds(i * 16, 16))
      mg = uv > ustar
      me = uv == ustar
      cg = plsc.cumsum(jnp.where(mg, one, zero))
      ce = plsc.cumsum(jnp.where(me, one, zero))
      posg = pre_g + rg + cg - one
      grank = pre_e + re + ce - one
      take = me & (grank < mvec)
      pose = tot_g + grank
      pos = jnp.where(mg, posg, pose)
      sel = mg | take
      gidx = gbase + jnp.full((16,), i * 16, jnp.int32)
      plsc.store_scatter(buf_v, [pos], gidx, mask=sel)
      return (rg + cg[15], re + ce[15])

    lax.fori_loop(0, nsl, pbody, (zero, zero))

    pltpu.sync_copy(buf_v, shb.at[pl.ds(sid * IDXPAD, IDXPAD)])
    plsc.subcore_barrier()
    obase = sid * och

    def z2body(i, _):
      acc_v[pl.ds(i * 16, 16)] = zero
      return 0

    lax.fori_loop(0, och // 16, z2body, 0)
    for t in range(16):
      pltpu.sync_copy(shb.at[pl.ds(t * IDXPAD + obase, och)], tmp_v)
      def abody(i, _):
        acc_v[pl.ds(i * 16, 16)] = (
            acc_v[pl.ds(i * 16, 16)] + tmp_v[pl.ds(i * 16, 16)])
        return 0
      lax.fori_loop(0, och // 16, abody, 0)
    pltpu.sync_copy(acc_v, idx_hbm.at[pl.ds(n * IDXPAD + obase, och)])
    plsc.subcore_barrier()

  return k


def _make_gather(hw_side, shift, wtab):
  ppt = NPT // 16
  nch = ppt // 128
  mesh = plsc.VectorSubcoreMesh(core_axis_name="c", subcore_axis_name="s", num_cores=2, num_subcores=16)
  w1tab = [jnp.float32(x) for x in wtab]
  mask_lo = (1 << shift) - 1

  @functools.partial(
      pl.kernel,
      out_type=jax.ShapeDtypeStruct((2, NPT, 128), jnp.float32),
      mesh=mesh,
      compiler_params=pltpu.CompilerParams(needs_layout_passes=False),
      interpret=_INTERP,
      scratch_types=[
          pltpu.VMEM((ppt,), jnp.int32),
          pltpu.VMEM((4, 128), jnp.int32),
          pltpu.VMEM((128, 16), jnp.float32),
          pltpu.VMEM((4, 128, 128), jnp.float32),
          pltpu.VMEM((128, 128), jnp.float32),
          pltpu.SemaphoreType.DMA,
      ],
  )
  def k(idx_hbm, tab_hbm, pts_hbm, idx_v, ti_v, tw_v, rows_v, out_v, sem):
    n = lax.axis_index("c")
    sid = lax.axis_index("s")
    pbase = sid * ppt
    pltpu.sync_copy(idx_hbm.at[pl.ds(n * IDXPAD + pbase, ppt)], idx_v)

    zero = jnp.zeros((16,), jnp.float32)

    def w1_of(r):
      w = jnp.full((16,), w1tab[0], jnp.float32)
      for j in range(1, len(w1tab)):
        w = jnp.where(r == j, jnp.full((16,), w1tab[j], jnp.float32), w)
      return w

    for chnk in range(nch):
      for sl in range(8):
        pix = idx_v[pl.ds(chnk * 128 + sl * 16, 16)]
        ix = pix & jnp.int32(hw_side - 1)
        iy = pix >> jnp.int32(hw_side.bit_length() - 1)
        x0 = (ix - jnp.int32(1 << (shift - 1))) >> jnp.int32(shift)
        y0 = (iy - jnp.int32(1 << (shift - 1))) >> jnp.int32(shift)
        wx1 = w1_of(ix & jnp.int32(mask_lo))
        wy1 = w1_of(iy & jnp.int32(mask_lo))
        wx0 = 1.0 - wx1
        wy0 = 1.0 - wy1
        x1 = x0 + 1
        y1 = y0 + 1
        vx0 = x0 >= 0
        vx1 = x1 <= 127
        vy0 = y0 >= 0
        vy1 = y1 <= 127
        xc0 = jnp.maximum(x0, 0)
        xc1 = jnp.minimum(x1, 127)
        yc0 = jnp.maximum(y0, 0)
        yc1 = jnp.minimum(y1, 127)
        taps = [
            (yc0, xc0, jnp.where(vy0 & vx0, wy0 * wx0, zero)),
            (yc0, xc1, jnp.where(vy0 & vx1, wy0 * wx1, zero)),
            (yc1, xc0, jnp.where(vy1 & vx0, wy1 * wx0, zero)),
            (yc1, xc1, jnp.where(vy1 & vx1, wy1 * wx1, zero)),
        ]
        rowi = jnp.full((16,), sl * 16, jnp.int32) + lax.iota(jnp.int32, 16)
        for t, (yy, xx, ww) in enumerate(taps):
          ti_v[t, pl.ds(sl * 16, 16)] = yy * 128 + xx
          plsc.store_scatter(
              tw_v, [rowi, jnp.full((16,), t, jnp.int32)], ww)

      cps = [
          pltpu.async_copy(tab_hbm.at[n].at[ti_v.at[t]], rows_v.at[t], sem)
          for t in range(4)
      ]
      for cp in cps:
        cp.wait()

      def comb(p, _):
        wrow = tw_v[p, :]
        w0 = jnp.full((16,), wrow[0], jnp.float32)
        w1 = jnp.full((16,), wrow[1], jnp.float32)
        w2 = jnp.full((16,), wrow[2], jnp.float32)
        w3 = jnp.full((16,), wrow[3], jnp.float32)
        for kk in range(8):
          s = pl.ds(kk * 16, 16)
          acc = (w0 * rows_v[0, p, s] + w1 * rows_v[1, p, s]
                 + w2 * rows_v[2, p, s] + w3 * rows_v[3, p, s])
          out_v[p, s] = acc
        return 0

      lax.fori_loop(0, 128, comb, 0)
      pltpu.sync_copy(out_v, pts_hbm.at[n, pl.ds(pbase + chnk * 128, 128), :])

  return k


def _k4_body(x_ref, w1_ref, w2h_ref, w2c_ref, w3h_ref, w3c_ref, wph_ref,
             wpc_ref, b1_ref, b2_ref, b3_ref, bp_ref, o_ref):
  bf = jnp.bfloat16

  def dot(a, w):
    return jnp.dot(a.astype(bf), w.astype(bf),
                   preferred_element_type=jnp.float32)

  x0 = x_ref[0]
  h = jnp.maximum(dot(x0, w1_ref[...]) + b1_ref[0:1, :], 0.0)
  h = jnp.maximum(dot(h, w2h_ref[...]) + dot(x0, w2c_ref[...])
                  + b2_ref[0:1, :], 0.0)
  h = jnp.maximum(dot(h, w3h_ref[...]) + dot(x0, w3c_ref[...])
                  + b3_ref[0:1, :], 0.0)
  o = (dot(h, wph_ref[...]) + dot(x0, wpc_ref[...]) + bp_ref[0:1, :])
  o_ref[0] = o


def _k4(pts, wd):
  n = pts.shape[0]
  pb = 2048
  npb = NPT // pb
  wspecs = [pl.BlockSpec(w.shape, lambda i, j: tuple([0] * w.ndim))
            for w in wd]
  return pl.pallas_call(
      _k4_body,
      interpret=_INTERP,
      grid=(n, npb),
      in_specs=[pl.BlockSpec((1, pb, 128), lambda i, j: (i, j, 0))] + wspecs,
      out_specs=pl.BlockSpec((1, pb, 32), lambda i, j: (i, j, 0)),
      out_shape=jax.ShapeDtypeStruct((n, NPT, 32), jnp.float32),
  )(pts, *wd)


def _make_scatter(hw):
  cpix = hw // 16
  ppt = NPT // 16
  mesh = plsc.VectorSubcoreMesh(core_axis_name="c", subcore_axis_name="s", num_cores=2, num_subcores=16)

  @functools.partial(
      pl.kernel,
      out_type=jax.ShapeDtypeStruct((2 * NCLS * hw,), jnp.float32),
      mesh=mesh,
      compiler_params=pltpu.CompilerParams(needs_layout_passes=False),
      interpret=_INTERP,
      scratch_types=[
          pltpu.VMEM((ppt,), jnp.int32),
          pltpu.VMEM((4, 128), jnp.int32),
          pltpu.VMEM((ppt,), jnp.float32),
          pltpu.VMEM((4, 128), jnp.float32),
          pltpu.SemaphoreType.DMA,
      ],
  )
  def k(up_hbm, idx_hbm, plt_hbm, out_hbm, idx_v, idxc_v, valf_v, val_v, sem):
    n = lax.axis_index("c")
    sid = lax.axis_index("s")
    for c in range(NCLS):
      cb = (n * NCLS + c) * hw + sid * cpix
      pltpu.sync_copy(up_hbm.at[pl.ds(cb, cpix)], out_hbm.at[pl.ds(cb, cpix)])
    pltpu.sync_copy(idx_hbm.at[pl.ds(n * IDXPAD + sid * ppt, ppt)], idx_v)
    plsc.subcore_barrier()
    for c in range(NCLS):
      pltpu.sync_copy(
          plt_hbm.at[pl.ds((n * 32 + c) * NPT + sid * ppt, ppt)], valf_v)
      cbase = jnp.full((16,), (n * NCLS + c) * hw, jnp.int32)
      for j in range(4):
        for kk in range(8):
          s = pl.ds(j * 128 + kk * 16, 16)
          s2 = pl.ds(kk * 16, 16)
          idxc_v[j, s2] = idx_v[s] + cbase
          val_v[j, s2] = valf_v[s]
      cps = [
          pltpu.async_copy(val_v.at[j], out_hbm.at[idxc_v.at[j]], sem)
          for j in range(4)
      ]
      for cp in cps:
        cp.wait()

  return k


def kernel(features, coarse_logits, W1, b1, W2, b2, W3, b3, Wp, bp):
  n, f, h0, w0 = features.shape
  c = coarse_logits.shape[1]

  tab = jnp.concatenate([features, coarse_logits], axis=1)
  tab = tab.reshape(n, f + c, h0 * w0).transpose(0, 2, 1)
  tab = jnp.pad(tab, ((0, 0), (0, 0), (0, 128 - (f + c))))

  w1t = jnp.zeros((128, FDIM), jnp.float32).at[: f + c].set(W1.T)
  w2h = W2[:, :FDIM].T
  w2c = jnp.zeros((128, FDIM), jnp.float32).at[f : f + c].set(W2[:, FDIM:].T)
  w3h = W3[:, :FDIM].T
  w3c = jnp.zeros((128, FDIM), jnp.float32).at[f : f + c].set(W3[:, FDIM:].T)
  wph = jnp.zeros((FDIM, 32), jnp.float32).at[:, :c].set(Wp[:, :FDIM].T)
  wpc = jnp.zeros((128, 32), jnp.float32).at[f : f + c, :c].set(Wp[:, FDIM:].T)
  b1r = jnp.broadcast_to(b1, (8, FDIM))
  b2r = jnp.broadcast_to(b2, (8, FDIM))
  b3r = jnp.broadcast_to(b3, (8, FDIM))
  bpr = jnp.zeros((8, 32), jnp.float32).at[0, :c].set(bp)
  wd = [w1t, w2h, w2c, w3h, w3c, wph, wpc, b1r, b2r, b3r, bpr]

  sem = coarse_logits
  for step in range(2):
    hin = sem.shape[2]
    h2 = 2 * hin
    hw = h2 * h2
    uw = _upmat(hin)
    uh = uw.T
    xw = _k1a(sem, uw)
    up, key = _k1b(xw, uh)
    idx = _make_topk(hw)(key.reshape(n * hw))
    shift = 1 if step == 0 else 2
    wtab = (0.75, 0.25) if step == 0 else (0.625, 0.875, 0.125, 0.375)
    pts = _make_gather(h2, shift, wtab)(idx, tab)
    po = _k4(pts, wd)
    plt = jnp.swapaxes(po, 1, 2).reshape(n * 32 * NPT)
    out = _make_scatter(hw)(up.reshape(n * c * hw), idx, plt)
    sem = out.reshape(n, c, h2, h2)
  return sem


# speedup vs baseline: 4.1845x; 1.9270x over previous
import functools

import jax
import jax.numpy as jnp
import numpy as np
from jax import lax
from jax.experimental import pallas as pl
from jax.experimental.pallas import tpu as pltpu
from jax.experimental.pallas import tpu_sc as plsc

NCLS = 19
_INTERP = False
NPT = 8192
FDIM = 256
IDXPAD = NPT + 256


def _upmat(h):
  o = np.arange(2 * h)
  coord = o * 0.5 - 0.25
  i0 = np.floor(coord).astype(np.int64)
  w1 = (coord - i0).astype(np.float32)
  u = np.zeros((h, 2 * h), np.float32)
  np.add.at(u, (np.clip(i0, 0, h - 1), o), 1.0 - w1)
  np.add.at(u, (np.clip(i0 + 1, 0, h - 1), o), w1)
  return jnp.asarray(u)


def _k1a_body(x_ref, uw_ref, o_ref):
  x = x_ref[0, 0]
  o_ref[0, 0] = jnp.dot(x, uw_ref[...], preferred_element_type=jnp.float32,
                        precision=lax.Precision.HIGHEST)


def _k1a(sem, uw):
  n, c, h, w = sem.shape
  return pl.pallas_call(
      _k1a_body,
      interpret=_INTERP,
      grid=(n, c),
      in_specs=[
          pl.BlockSpec((1, 1, h, w), lambda i, j: (i, j, 0, 0)),
          pl.BlockSpec((w, 2 * w), lambda i, j: (0, 0)),
      ],
      out_specs=pl.BlockSpec((1, 1, h, 2 * w), lambda i, j: (i, j, 0, 0)),
      out_shape=jax.ShapeDtypeStruct((n, c, h, 2 * w), jnp.float32),
  )(sem, uw)


def _k1b_body(x_ref, uh_ref, up_ref, key_ref):
  uh = uh_ref[...]
  m1 = None
  m2 = None
  for c in range(NCLS):
    u = jnp.dot(uh, x_ref[0, c], preferred_element_type=jnp.float32,
                precision=lax.Precision.HIGHEST)
    up_ref[0, c] = u
    if c == 0:
      m1 = u
      m2 = jnp.full_like(u, -jnp.inf)
    else:
      nm1 = jnp.maximum(m1, u)
      m2 = jnp.maximum(m2, jnp.minimum(m1, u))
      m1 = nm1
  unc = m2 - m1
  b = lax.bitcast_convert_type(unc, jnp.int32)
  key_ref[0] = jnp.where(b < 0, b ^ jnp.int32(0x7FFFFFFF), b)


def _k1b(xw, uh):
  n, c, h, w2 = xw.shape
  h2 = 2 * h
  wt = 128
  nw = w2 // wt
  return pl.pallas_call(
      _k1b_body,
      interpret=_INTERP,
      grid=(n, nw),
      in_specs=[
          pl.BlockSpec((1, c, h, wt), lambda i, j: (i, 0, 0, j)),
          pl.BlockSpec((h2, h), lambda i, j: (0, 0)),
      ],
      out_specs=[
          pl.BlockSpec((1, c, h2, wt), lambda i, j: (i, 0, 0, j)),
          pl.BlockSpec((1, h2, wt), lambda i, j: (i, 0, j)),
      ],
      out_shape=[
          jax.ShapeDtypeStruct((n, c, h2, w2), jnp.float32),
          jax.ShapeDtypeStruct((n, h2, w2), jnp.int32),
      ],
  )(xw, uh)


def _make_topk(hw):
  ch = hw // 16
  nsl = ch // 16
  och = IDXPAD // 16
  mesh = plsc.VectorSubcoreMesh(core_axis_name="c", subcore_axis_name="s", num_cores=2, num_subcores=16)

  @functools.partial(
      pl.kernel,
      out_type=jax.ShapeDtypeStruct((2 * IDXPAD,), jnp.int32),
      mesh=mesh,
      compiler_params=pltpu.CompilerParams(needs_layout_passes=False),
      interpret=_INTERP,
      scratch_types=[
          pltpu.VMEM((ch,), jnp.int32),
          pltpu.VMEM((16,), jnp.int32),
          pltpu.VMEM((256,), jnp.int32),
          pltpu.VMEM((IDXPAD,), jnp.int32),
          pltpu.VMEM((och,), jnp.int32),
          pltpu.VMEM((och,), jnp.int32),
          pltpu.VMEM_SHARED((768,), jnp.int32),
          pltpu.VMEM_SHARED((16 * IDXPAD,), jnp.int32),
      ],
  )
  def k(keys_hbm, idx_hbm, keys_v, cnt_v, all_v, buf_v, acc_v, tmp_v, shc, shb):
    n = lax.axis_index("c")
    sid = lax.axis_index("s")
    sid16 = jnp.full((16,), sid, jnp.int32)
    base = sid * ch
    pltpu.sync_copy(keys_hbm.at[pl.ds(n * (16 * ch) + base, ch)], keys_v)

    kvec = jnp.full((16,), NPT, jnp.int32)
    zero = jnp.zeros((16,), jnp.int32)
    one = jnp.full((16,), 1, jnp.int32)
    lanes = lax.iota(jnp.int32, 16)
    u32 = jnp.uint32

    def tbody(i, _):
      s = pl.ds(i * 16, 16)
      v = lax.bitcast_convert_type(keys_v[s], u32) ^ u32(0x80000000)
      keys_v[s] = lax.bitcast_convert_type(v, jnp.int32)
      return 0

    lax.fori_loop(0, nsl, tbody, 0)

    def uload(s):
      return lax.bitcast_convert_type(keys_v[s], u32)

    def exchange(cnt_vec, slot):
      sb = (slot % 3) * 256
      cnt_v[...] = cnt_vec
      pltpu.sync_copy(cnt_v, shc.at[pl.ds(sb + sid * 16, 16)])
      plsc.subcore_barrier()
      pltpu.sync_copy(shc.at[pl.ds(sb, 256)], all_v)
      tot = zero
      pre = zero
      for j in range(16):
        row = all_v[pl.ds(j * 16, 16)]
        tot = tot + row
        pre = pre + jnp.where(jnp.full((16,), j, jnp.int32) < sid16, row, zero)
      return tot, pre

    p = jnp.full((16,), 0, u32)
    for r in range(8):
      shift = 28 - 4 * r
      ts = [p + u32(j << shift) for j in range(1, 16)]

      def rbody(i, accs):
        uv = uload(pl.ds(i * 16, 16))
        return tuple(a + jnp.where(uv >= t, one, zero)
                     for a, t in zip(accs, ts))

      accs = lax.fori_loop(0, nsl, rbody, (zero,) * 15)
      cvec = zero
      for j in range(15):
        cvec = jnp.where(lanes == j + 1,
                         jnp.full((16,), jnp.sum(accs[j]), jnp.int32), cvec)
      tot, _ = exchange(cvec, r)
      s = jnp.sum(jnp.where(tot >= kvec, one, zero))
      p = p + lax.convert_element_type(
          jnp.full((16,), lax.shift_left(s, shift), jnp.int32), u32)

    ustar = p

    def cbody(i, carry):
      ag, ae = carry
      uv = uload(pl.ds(i * 16, 16))
      ag = ag + jnp.where(uv > ustar, one, zero)
      ae = ae + jnp.where(uv == ustar, one, zero)
      return (ag, ae)

    ag, ae = lax.fori_loop(0, nsl, cbody, (zero, zero))
    c2 = jnp.where(lanes == 0, jnp.full((16,), jnp.sum(ag), jnp.int32),
                   jnp.where(lanes == 1,
                             jnp.full((16,), jnp.sum(ae), jnp.int32), zero))
    tot2, pre2 = exchange(c2, 8)
    tot_g = jnp.full((16,), tot2[0], jnp.int32)
    pre_g = jnp.full((16,), pre2[0], jnp.int32)
    pre_e = jnp.full((16,), pre2[1], jnp.int32)
    mvec = kvec - tot_g

    def zbody(i, _):
      buf_v[pl.ds(i * 16, 16)] = zero
      return 0

    lax.fori_loop(0, IDXPAD // 16, zbody, 0)

    gbase = jnp.full((16,), base, jnp.int32) + lax.iota(jnp.int32, 16)

    def pbody(i, carry):
      rg, re = carry
      uv = uload(pl.ds(i * 16, 16))
      mg = uv > ustar
      me = uv == ustar
      cg = plsc.cumsum(jnp.where(mg, one, zero))
      ce = plsc.cumsum(jnp.where(me, one, zero))
      posg = pre_g + rg + cg - one
      grank = pre_e + re + ce - one
      take = me & (grank < mvec)
      pose = tot_g + grank
      pos = jnp.where(mg, posg, pose)
      sel = mg | take
      gidx = gbase + jnp.full((16,), i * 16, jnp.int32)
      plsc.store_scatter(buf_v, [pos], gidx, mask=sel)
      return (rg + cg[15], re + ce[15])

    lax.fori_loop(0, nsl, pbody, (zero, zero))

    pltpu.sync_copy(buf_v, shb.at[pl.ds(sid * IDXPAD, IDXPAD)])
    plsc.subcore_barrier()
    obase = sid * och

    def z2body(i, _):
      acc_v[pl.ds(i * 16, 16)] = zero
      return 0

    lax.fori_loop(0, och // 16, z2body, 0)
    for t in range(16):
      pltpu.sync_copy(shb.at[pl.ds(t * IDXPAD + obase, och)], tmp_v)
      def abody(i, _):
        acc_v[pl.ds(i * 16, 16)] = (
            acc_v[pl.ds(i * 16, 16)] + tmp_v[pl.ds(i * 16, 16)])
        return 0
      lax.fori_loop(0, och // 16, abody, 0)
    pltpu.sync_copy(acc_v, idx_hbm.at[pl.ds(n * IDXPAD + obase, och)])
    plsc.subcore_barrier()

  return k


def _make_gather(hw_side, shift, wtab):
  ppt = NPT // 16
  nch = ppt // 128
  mesh = plsc.VectorSubcoreMesh(core_axis_name="c", subcore_axis_name="s", num_cores=2, num_subcores=16)
  w1tab = [jnp.float32(x) for x in wtab]
  mask_lo = (1 << shift) - 1

  @functools.partial(
      pl.kernel,
      out_type=jax.ShapeDtypeStruct((2, NPT, 128), jnp.float32),
      mesh=mesh,
      compiler_params=pltpu.CompilerParams(needs_layout_passes=False),
      interpret=_INTERP,
      scratch_types=[
          pltpu.VMEM((ppt,), jnp.int32),
          pltpu.VMEM((4, 128), jnp.int32),
          pltpu.VMEM((128, 16), jnp.float32),
          pltpu.VMEM((4, 128, 128), jnp.float32),
          pltpu.VMEM((128, 128), jnp.float32),
          pltpu.SemaphoreType.DMA,
      ],
  )
  def k(idx_hbm, tab_hbm, pts_hbm, idx_v, ti_v, tw_v, rows_v, out_v, sem):
    n = lax.axis_index("c")
    sid = lax.axis_index("s")
    pbase = sid * ppt
    pltpu.sync_copy(idx_hbm.at[pl.ds(n * IDXPAD + pbase, ppt)], idx_v)

    zero = jnp.zeros((16,), jnp.float32)

    def w1_of(r):
      w = jnp.full((16,), w1tab[0], jnp.float32)
      for j in range(1, len(w1tab)):
        w = jnp.where(r == j, jnp.full((16,), w1tab[j], jnp.float32), w)
      return w

    for chnk in range(nch):
      for sl in range(8):
        pix = idx_v[pl.ds(chnk * 128 + sl * 16, 16)]
        ix = pix & jnp.int32(hw_side - 1)
        iy = pix >> jnp.int32(hw_side.bit_length() - 1)
        x0 = (ix - jnp.int32(1 << (shift - 1))) >> jnp.int32(shift)
        y0 = (iy - jnp.int32(1 << (shift - 1))) >> jnp.int32(shift)
        wx1 = w1_of(ix & jnp.int32(mask_lo))
        wy1 = w1_of(iy & jnp.int32(mask_lo))
        wx0 = 1.0 - wx1
        wy0 = 1.0 - wy1
        x1 = x0 + 1
        y1 = y0 + 1
        vx0 = x0 >= 0
        vx1 = x1 <= 127
        vy0 = y0 >= 0
        vy1 = y1 <= 127
        xc0 = jnp.maximum(x0, 0)
        xc1 = jnp.minimum(x1, 127)
        yc0 = jnp.maximum(y0, 0)
        yc1 = jnp.minimum(y1, 127)
        taps = [
            (yc0, xc0, jnp.where(vy0 & vx0, wy0 * wx0, zero)),
            (yc0, xc1, jnp.where(vy0 & vx1, wy0 * wx1, zero)),
            (yc1, xc0, jnp.where(vy1 & vx0, wy1 * wx0, zero)),
            (yc1, xc1, jnp.where(vy1 & vx1, wy1 * wx1, zero)),
        ]
        rowi = jnp.full((16,), sl * 16, jnp.int32) + lax.iota(jnp.int32, 16)
        for t, (yy, xx, ww) in enumerate(taps):
          ti_v[t, pl.ds(sl * 16, 16)] = yy * 128 + xx
          plsc.store_scatter(
              tw_v, [rowi, jnp.full((16,), t, jnp.int32)], ww)

      cps = [
          pltpu.async_copy(tab_hbm.at[n].at[ti_v.at[t]], rows_v.at[t], sem)
          for t in range(4)
      ]
      for cp in cps:
        cp.wait()

      def comb(p, _):
        wrow = tw_v[p, :]
        w0 = jnp.full((16,), wrow[0], jnp.float32)
        w1 = jnp.full((16,), wrow[1], jnp.float32)
        w2 = jnp.full((16,), wrow[2], jnp.float32)
        w3 = jnp.full((16,), wrow[3], jnp.float32)
        for kk in range(8):
          s = pl.ds(kk * 16, 16)
          acc = (w0 * rows_v[0, p, s] + w1 * rows_v[1, p, s]
                 + w2 * rows_v[2, p, s] + w3 * rows_v[3, p, s])
          out_v[p, s] = acc
        return 0

      lax.fori_loop(0, 128, comb, 0)
      pltpu.sync_copy(out_v, pts_hbm.at[n, pl.ds(pbase + chnk * 128, 128), :])

  return k


def _k4_body(x_ref, w1_ref, w2h_ref, w2c_ref, w3h_ref, w3c_ref, wph_ref,
             wpc_ref, b1_ref, b2_ref, b3_ref, bp_ref, o_ref):
  bf = jnp.bfloat16

  def dot(a, w):
    return jnp.dot(a.astype(bf), w.astype(bf),
                   preferred_element_type=jnp.float32)

  x0 = x_ref[0]
  h = jnp.maximum(dot(x0, w1_ref[...]) + b1_ref[0:1, :], 0.0)
  h = jnp.maximum(dot(h, w2h_ref[...]) + dot(x0, w2c_ref[...])
                  + b2_ref[0:1, :], 0.0)
  h = jnp.maximum(dot(h, w3h_ref[...]) + dot(x0, w3c_ref[...])
                  + b3_ref[0:1, :], 0.0)
  o = (dot(h, wph_ref[...]) + dot(x0, wpc_ref[...]) + bp_ref[0:1, :])
  o_ref[0] = o


def _k4(pts, wd):
  n = pts.shape[0]
  pb = 2048
  npb = NPT // pb
  wspecs = [pl.BlockSpec(w.shape, lambda i, j: tuple([0] * w.ndim))
            for w in wd]
  return pl.pallas_call(
      _k4_body,
      interpret=_INTERP,
      grid=(n, npb),
      in_specs=[pl.BlockSpec((1, pb, 128), lambda i, j: (i, j, 0))] + wspecs,
      out_specs=pl.BlockSpec((1, pb, 32), lambda i, j: (i, j, 0)),
      out_shape=jax.ShapeDtypeStruct((n, NPT, 32), jnp.float32),
  )(pts, *wd)


def _make_scatter(hw):
  cpix = hw // 16
  ppt = NPT // 16
  mesh = plsc.VectorSubcoreMesh(core_axis_name="c", subcore_axis_name="s", num_cores=2, num_subcores=16)

  @functools.partial(
      pl.kernel,
      out_type=jax.ShapeDtypeStruct((2 * NCLS * hw,), jnp.float32),
      mesh=mesh,
      compiler_params=pltpu.CompilerParams(needs_layout_passes=False),
      interpret=_INTERP,
      scratch_types=[
          pltpu.VMEM((ppt,), jnp.int32),
          pltpu.VMEM((4, 128), jnp.int32),
          pltpu.VMEM((ppt,), jnp.float32),
          pltpu.VMEM((4, 128), jnp.float32),
          pltpu.VMEM((cpix,), jnp.float32),
          pltpu.VMEM((cpix,), jnp.float32),
          pltpu.SemaphoreType.DMA,
          pltpu.SemaphoreType.DMA,
          pltpu.SemaphoreType.DMA,
      ],
  )
  def k(up_hbm, idx_hbm, plt_hbm, out_hbm, idx_v, idxc_v, valf_v, val_v,
        cb_a, cb_b, sem, lsem, ssem):
    n = lax.axis_index("c")
    sid = lax.axis_index("s")
    bufs = [cb_a, cb_b]
    st = [None, None]

    def cbase(c):
      return (n * NCLS + c) * hw + sid * cpix

    for c in range(NCLS):
      b = c % 2
      if st[b] is not None:
        st[b].wait()
      pltpu.async_copy(up_hbm.at[pl.ds(cbase(c), cpix)], bufs[b], lsem).wait()
      st[b] = pltpu.async_copy(bufs[b], out_hbm.at[pl.ds(cbase(c), cpix)],
                               ssem)
    st[0].wait()
    st[1].wait()
    pltpu.sync_copy(idx_hbm.at[pl.ds(n * IDXPAD + sid * ppt, ppt)], idx_v)
    plsc.subcore_barrier()
    for c in range(NCLS):
      pltpu.sync_copy(
          plt_hbm.at[pl.ds((n * 32 + c) * NPT + sid * ppt, ppt)], valf_v)
      cbase = jnp.full((16,), (n * NCLS + c) * hw, jnp.int32)
      for j in range(4):
        for kk in range(8):
          s = pl.ds(j * 128 + kk * 16, 16)
          s2 = pl.ds(kk * 16, 16)
          idxc_v[j, s2] = idx_v[s] + cbase
          val_v[j, s2] = valf_v[s]
      cps = [
          pltpu.async_copy(val_v.at[j], out_hbm.at[idxc_v.at[j]], sem)
          for j in range(4)
      ]
      for cp in cps:
        cp.wait()

  return k


def kernel(features, coarse_logits, W1, b1, W2, b2, W3, b3, Wp, bp):
  n, f, h0, w0 = features.shape
  c = coarse_logits.shape[1]

  tab = jnp.concatenate([features, coarse_logits], axis=1)
  tab = tab.reshape(n, f + c, h0 * w0).transpose(0, 2, 1)
  tab = jnp.pad(tab, ((0, 0), (0, 0), (0, 128 - (f + c))))

  w1t = jnp.zeros((128, FDIM), jnp.float32).at[: f + c].set(W1.T)
  w2h = W2[:, :FDIM].T
  w2c = jnp.zeros((128, FDIM), jnp.float32).at[f : f + c].set(W2[:, FDIM:].T)
  w3h = W3[:, :FDIM].T
  w3c = jnp.zeros((128, FDIM), jnp.float32).at[f : f + c].set(W3[:, FDIM:].T)
  wph = jnp.zeros((FDIM, 32), jnp.float32).at[:, :c].set(Wp[:, :FDIM].T)
  wpc = jnp.zeros((128, 32), jnp.float32).at[f : f + c, :c].set(Wp[:, FDIM:].T)
  b1r = jnp.broadcast_to(b1, (8, FDIM))
  b2r = jnp.broadcast_to(b2, (8, FDIM))
  b3r = jnp.broadcast_to(b3, (8, FDIM))
  bpr = jnp.zeros((8, 32), jnp.float32).at[0, :c].set(bp)
  wd = [w1t, w2h, w2c, w3h, w3c, wph, wpc, b1r, b2r, b3r, bpr]

  sem = coarse_logits
  for step in range(2):
    hin = sem.shape[2]
    h2 = 2 * hin
    hw = h2 * h2
    uw = _upmat(hin)
    uh = uw.T
    xw = _k1a(sem, uw)
    up, key = _k1b(xw, uh)
    idx = _make_topk(hw)(key.reshape(n * hw))
    shift = 1 if step == 0 else 2
    wtab = (0.75, 0.25) if step == 0 else (0.625, 0.875, 0.125, 0.375)
    pts = _make_gather(h2, shift, wtab)(idx, tab)
    po = _k4(pts, wd)
    plt = jnp.swapaxes(po, 1, 2).reshape(n * 32 * NPT)
    out = _make_scatter(hw)(up.reshape(n * c * hw), idx, plt)
    sem = out.reshape(n, c, h2, h2)
  return sem


# speedup vs baseline: 4.1957x; 1.0027x over previous
import functools

import jax
import jax.numpy as jnp
import numpy as np
from jax import lax
from jax.experimental import pallas as pl
from jax.experimental.pallas import tpu as pltpu
from jax.experimental.pallas import tpu_sc as plsc

NCLS = 19
_INTERP = False
NPT = 8192
FDIM = 256
IDXPAD = NPT + 256


def _upmat(h):
  o = np.arange(2 * h)
  coord = o * 0.5 - 0.25
  i0 = np.floor(coord).astype(np.int64)
  w1 = (coord - i0).astype(np.float32)
  u = np.zeros((h, 2 * h), np.float32)
  np.add.at(u, (np.clip(i0, 0, h - 1), o), 1.0 - w1)
  np.add.at(u, (np.clip(i0 + 1, 0, h - 1), o), w1)
  return jnp.asarray(u)


def _k1a_body(x_ref, uw_ref, o_ref):
  x = x_ref[0, 0]
  o_ref[0, 0] = jnp.dot(x, uw_ref[...], preferred_element_type=jnp.float32,
                        precision=lax.Precision.HIGHEST)


def _k1a(sem, uw):
  n, c, h, w = sem.shape
  return pl.pallas_call(
      _k1a_body,
      interpret=_INTERP,
      grid=(n, c),
      in_specs=[
          pl.BlockSpec((1, 1, h, w), lambda i, j: (i, j, 0, 0)),
          pl.BlockSpec((w, 2 * w), lambda i, j: (0, 0)),
      ],
      out_specs=pl.BlockSpec((1, 1, h, 2 * w), lambda i, j: (i, j, 0, 0)),
      out_shape=jax.ShapeDtypeStruct((n, c, h, 2 * w), jnp.float32),
  )(sem, uw)


def _k1b_body(x_ref, uh_ref, up_ref, key_ref):
  uh = uh_ref[...]
  m1 = None
  m2 = None
  for c in range(NCLS):
    u = jnp.dot(uh, x_ref[0, c], preferred_element_type=jnp.float32,
                precision=lax.Precision.HIGHEST)
    up_ref[0, c] = u
    if c == 0:
      m1 = u
      m2 = jnp.full_like(u, -jnp.inf)
    else:
      nm1 = jnp.maximum(m1, u)
      m2 = jnp.maximum(m2, jnp.minimum(m1, u))
      m1 = nm1
  unc = m2 - m1
  b = lax.bitcast_convert_type(unc, jnp.int32)
  key_ref[0] = jnp.where(b < 0, b ^ jnp.int32(0x7FFFFFFF), b)


def _k1b(xw, uh):
  n, c, h, w2 = xw.shape
  h2 = 2 * h
  wt = 128
  nw = w2 // wt
  return pl.pallas_call(
      _k1b_body,
      interpret=_INTERP,
      grid=(n, nw),
      in_specs=[
          pl.BlockSpec((1, c, h, wt), lambda i, j: (i, 0, 0, j)),
          pl.BlockSpec((h2, h), lambda i, j: (0, 0)),
      ],
      out_specs=[
          pl.BlockSpec((1, c, h2, wt), lambda i, j: (i, 0, 0, j)),
          pl.BlockSpec((1, h2, wt), lambda i, j: (i, 0, j)),
      ],
      out_shape=[
          jax.ShapeDtypeStruct((n, c, h2, w2), jnp.float32),
          jax.ShapeDtypeStruct((n, h2, w2), jnp.int32),
      ],
  )(xw, uh)


def _make_topk(hw):
  ch = hw // 16
  nsl = ch // 16
  och = IDXPAD // 16
  mesh = plsc.VectorSubcoreMesh(core_axis_name="c", subcore_axis_name="s", num_cores=2, num_subcores=16)

  @functools.partial(
      pl.kernel,
      out_type=jax.ShapeDtypeStruct((2 * IDXPAD,), jnp.int32),
      mesh=mesh,
      compiler_params=pltpu.CompilerParams(needs_layout_passes=False),
      interpret=_INTERP,
      scratch_types=[
          pltpu.VMEM((ch,), jnp.int32),
          pltpu.VMEM((16,), jnp.int32),
          pltpu.VMEM((256,), jnp.int32),
          pltpu.VMEM((IDXPAD,), jnp.int32),
          pltpu.VMEM((och,), jnp.int32),
          pltpu.VMEM((och,), jnp.int32),
          pltpu.VMEM_SHARED((768,), jnp.int32),
          pltpu.VMEM_SHARED((16 * IDXPAD,), jnp.int32),
      ],
  )
  def k(keys_hbm, idx_hbm, keys_v, cnt_v, all_v, buf_v, acc_v, tmp_v, shc, shb):
    n = lax.axis_index("c")
    sid = lax.axis_index("s")
    sid16 = jnp.full((16,), sid, jnp.int32)
    base = sid * ch
    pltpu.sync_copy(keys_hbm.at[pl.ds(n * (16 * ch) + base, ch)], keys_v)

    kvec = jnp.full((16,), NPT, jnp.int32)
    zero = jnp.zeros((16,), jnp.int32)
    one = jnp.full((16,), 1, jnp.int32)
    lanes = lax.iota(jnp.int32, 16)
    u32 = jnp.uint32

    def tbody(i, _):
      s = pl.ds(i * 16, 16)
      v = lax.bitcast_convert_type(keys_v[s], u32) ^ u32(0x80000000)
      keys_v[s] = lax.bitcast_convert_type(v, jnp.int32)
      return 0

    lax.fori_loop(0, nsl, tbody, 0)

    def uload(s):
      return lax.bitcast_convert_type(keys_v[s], u32)

    def exchange(cnt_vec, slot):
      sb = (slot % 3) * 256
      cnt_v[...] = cnt_vec
      pltpu.sync_copy(cnt_v, shc.at[pl.ds(sb + sid * 16, 16)])
      plsc.subcore_barrier()
      pltpu.sync_copy(shc.at[pl.ds(sb, 256)], all_v)
      tot = zero
      pre = zero
      for j in range(16):
        row = all_v[pl.ds(j * 16, 16)]
        tot = tot + row
        pre = pre + jnp.where(jnp.full((16,), j, jnp.int32) < sid16, row, zero)
      return tot, pre

    p = jnp.full((16,), 0, u32)
    for r in range(8):
      shift = 28 - 4 * r
      ts = [p + u32(j << shift) for j in range(1, 16)]

      def rbody(i, accs):
        uv = uload(pl.ds(i * 16, 16))
        return tuple(a + jnp.where(uv >= t, one, zero)
                     for a, t in zip(accs, ts))

      accs = lax.fori_loop(0, nsl, rbody, (zero,) * 15)
      cvec = zero
      for j in range(15):
        cvec = jnp.where(lanes == j + 1,
                         jnp.full((16,), jnp.sum(accs[j]), jnp.int32), cvec)
      tot, _ = exchange(cvec, r)
      s = jnp.sum(jnp.where(tot >= kvec, one, zero))
      p = p + lax.convert_element_type(
          jnp.full((16,), lax.shift_left(s, shift), jnp.int32), u32)

    ustar = p

    def cbody(i, carry):
      ag, ae = carry
      uv = uload(pl.ds(i * 16, 16))
      ag = ag + jnp.where(uv > ustar, one, zero)
      ae = ae + jnp.where(uv == ustar, one, zero)
      return (ag, ae)

    ag, ae = lax.fori_loop(0, nsl, cbody, (zero, zero))
    c2 = jnp.where(lanes == 0, jnp.full((16,), jnp.sum(ag), jnp.int32),
                   jnp.where(lanes == 1,
                             jnp.full((16,), jnp.sum(ae), jnp.int32), zero))
    tot2, pre2 = exchange(c2, 8)
    tot_g = jnp.full((16,), tot2[0], jnp.int32)
    pre_g = jnp.full((16,), pre2[0], jnp.int32)
    pre_e = jnp.full((16,), pre2[1], jnp.int32)
    mvec = kvec - tot_g

    def zbody(i, _):
      buf_v[pl.ds(i * 16, 16)] = zero
      return 0

    lax.fori_loop(0, IDXPAD // 16, zbody, 0)

    gbase = jnp.full((16,), base, jnp.int32) + lax.iota(jnp.int32, 16)

    def pbody(i, carry):
      rg, re = carry
      uv = uload(pl.ds(i * 16, 16))
      mg = uv > ustar
      me = uv == ustar
      cg = plsc.cumsum(jnp.where(mg, one, zero))
      ce = plsc.cumsum(jnp.where(me, one, zero))
      posg = pre_g + rg + cg - one
      grank = pre_e + re + ce - one
      take = me & (grank < mvec)
      pose = tot_g + grank
      pos = jnp.where(mg, posg, pose)
      sel = mg | take
      gidx = gbase + jnp.full((16,), i * 16, jnp.int32)
      plsc.store_scatter(buf_v, [pos], gidx, mask=sel)
      return (rg + cg[15], re + ce[15])

    lax.fori_loop(0, nsl, pbody, (zero, zero))

    pltpu.sync_copy(buf_v, shb.at[pl.ds(sid * IDXPAD, IDXPAD)])
    plsc.subcore_barrier()
    obase = sid * och

    def z2body(i, _):
      acc_v[pl.ds(i * 16, 16)] = zero
      return 0

    lax.fori_loop(0, och // 16, z2body, 0)
    for t in range(16):
      pltpu.sync_copy(shb.at[pl.ds(t * IDXPAD + obase, och)], tmp_v)
      def abody(i, _):
        acc_v[pl.ds(i * 16, 16)] = (
            acc_v[pl.ds(i * 16, 16)] + tmp_v[pl.ds(i * 16, 16)])
        return 0
      lax.fori_loop(0, och // 16, abody, 0)
    pltpu.sync_copy(acc_v, idx_hbm.at[pl.ds(n * IDXPAD + obase, och)])
    plsc.subcore_barrier()

  return k


def _make_gather(hw_side, shift, wtab):
  ppt = NPT // 16
  nch = ppt // 128
  mesh = plsc.VectorSubcoreMesh(core_axis_name="c", subcore_axis_name="s", num_cores=2, num_subcores=16)
  w1tab = [jnp.float32(x) for x in wtab]
  mask_lo = (1 << shift) - 1

  @functools.partial(
      pl.kernel,
      out_type=jax.ShapeDtypeStruct((2, NPT, 128), jnp.float32),
      mesh=mesh,
      compiler_params=pltpu.CompilerParams(needs_layout_passes=False),
      interpret=_INTERP,
      scratch_types=[
          pltpu.VMEM((ppt,), jnp.int32),
          pltpu.VMEM((4, 128), jnp.int32),
          pltpu.VMEM((128, 16), jnp.float32),
          pltpu.VMEM((4, 128, 128), jnp.float32),
          pltpu.VMEM((128, 128), jnp.float32),
          pltpu.SemaphoreType.DMA,
      ],
  )
  def k(idx_hbm, tab_hbm, pts_hbm, idx_v, ti_v, tw_v, rows_v, out_v, sem):
    n = lax.axis_index("c")
    sid = lax.axis_index("s")
    pbase = sid * ppt
    pltpu.sync_copy(idx_hbm.at[pl.ds(n * IDXPAD + pbase, ppt)], idx_v)

    zero = jnp.zeros((16,), jnp.float32)

    def w1_of(r):
      w = jnp.full((16,), w1tab[0], jnp.float32)
      for j in range(1, len(w1tab)):
        w = jnp.where(r == j, jnp.full((16,), w1tab[j], jnp.float32), w)
      return w

    for chnk in range(nch):
      for sl in range(8):
        pix = idx_v[pl.ds(chnk * 128 + sl * 16, 16)]
        ix = pix & jnp.int32(hw_side - 1)
        iy = pix >> jnp.int32(hw_side.bit_length() - 1)
        x0 = (ix - jnp.int32(1 << (shift - 1))) >> jnp.int32(shift)
        y0 = (iy - jnp.int32(1 << (shift - 1))) >> jnp.int32(shift)
        wx1 = w1_of(ix & jnp.int32(mask_lo))
        wy1 = w1_of(iy & jnp.int32(mask_lo))
        wx0 = 1.0 - wx1
        wy0 = 1.0 - wy1
        x1 = x0 + 1
        y1 = y0 + 1
        vx0 = x0 >= 0
        vx1 = x1 <= 127
        vy0 = y0 >= 0
        vy1 = y1 <= 127
        xc0 = jnp.maximum(x0, 0)
        xc1 = jnp.minimum(x1, 127)
        yc0 = jnp.maximum(y0, 0)
        yc1 = jnp.minimum(y1, 127)
        taps = [
            (yc0, xc0, jnp.where(vy0 & vx0, wy0 * wx0, zero)),
            (yc0, xc1, jnp.where(vy0 & vx1, wy0 * wx1, zero)),
            (yc1, xc0, jnp.where(vy1 & vx0, wy1 * wx0, zero)),
            (yc1, xc1, jnp.where(vy1 & vx1, wy1 * wx1, zero)),
        ]
        rowi = jnp.full((16,), sl * 16, jnp.int32) + lax.iota(jnp.int32, 16)
        for t, (yy, xx, ww) in enumerate(taps):
          ti_v[t, pl.ds(sl * 16, 16)] = yy * 128 + xx
          plsc.store_scatter(
              tw_v, [rowi, jnp.full((16,), t, jnp.int32)], ww)

      cps = [
          pltpu.async_copy(tab_hbm.at[n].at[ti_v.at[t]], rows_v.at[t], sem)
          for t in range(4)
      ]
      for cp in cps:
        cp.wait()

      def comb(p, _):
        wrow = tw_v[p, :]
        w0 = jnp.full((16,), wrow[0], jnp.float32)
        w1 = jnp.full((16,), wrow[1], jnp.float32)
        w2 = jnp.full((16,), wrow[2], jnp.float32)
        w3 = jnp.full((16,), wrow[3], jnp.float32)
        for kk in range(8):
          s = pl.ds(kk * 16, 16)
          acc = (w0 * rows_v[0, p, s] + w1 * rows_v[1, p, s]
                 + w2 * rows_v[2, p, s] + w3 * rows_v[3, p, s])
          out_v[p, s] = acc
        return 0

      lax.fori_loop(0, 128, comb, 0)
      pltpu.sync_copy(out_v, pts_hbm.at[n, pl.ds(pbase + chnk * 128, 128), :])

  return k


def _k4_body(x_ref, w1_ref, w2h_ref, w2c_ref, w3h_ref, w3c_ref, wph_ref,
             wpc_ref, b1_ref, b2_ref, b3_ref, bp_ref, o_ref):
  bf = jnp.bfloat16

  def dot(a, w):
    return jnp.dot(a.astype(bf), w.astype(bf),
                   preferred_element_type=jnp.float32)

  x0 = x_ref[0]
  h = jnp.maximum(dot(x0, w1_ref[...]) + b1_ref[0:1, :], 0.0)
  h = jnp.maximum(dot(h, w2h_ref[...]) + dot(x0, w2c_ref[...])
                  + b2_ref[0:1, :], 0.0)
  h = jnp.maximum(dot(h, w3h_ref[...]) + dot(x0, w3c_ref[...])
                  + b3_ref[0:1, :], 0.0)
  o = (dot(h, wph_ref[...]) + dot(x0, wpc_ref[...]) + bp_ref[0:1, :])
  o_ref[0] = o


def _k4(pts, wd):
  n = pts.shape[0]
  pb = 2048
  npb = NPT // pb
  wspecs = [pl.BlockSpec(w.shape, lambda i, j: tuple([0] * w.ndim))
            for w in wd]
  return pl.pallas_call(
      _k4_body,
      interpret=_INTERP,
      grid=(n, npb),
      in_specs=[pl.BlockSpec((1, pb, 128), lambda i, j: (i, j, 0))] + wspecs,
      out_specs=pl.BlockSpec((1, pb, 32), lambda i, j: (i, j, 0)),
      out_shape=jax.ShapeDtypeStruct((n, NPT, 32), jnp.float32),
  )(pts, *wd)


def _make_scatter(hw):
  cpix = hw // 16
  ppt = NPT // 16
  mesh = plsc.VectorSubcoreMesh(core_axis_name="c", subcore_axis_name="s", num_cores=2, num_subcores=16)

  @functools.partial(
      pl.kernel,
      out_type=jax.ShapeDtypeStruct((2 * NCLS * hw,), jnp.float32),
      mesh=mesh,
      compiler_params=pltpu.CompilerParams(needs_layout_passes=False),
      interpret=_INTERP,
      scratch_types=[
          pltpu.VMEM((ppt,), jnp.int32),
          pltpu.VMEM((2, 4, 128), jnp.int32),
          pltpu.VMEM((2, ppt), jnp.float32),
          pltpu.VMEM((2, 4, 128), jnp.float32),
          pltpu.VMEM((cpix,), jnp.float32),
          pltpu.VMEM((cpix,), jnp.float32),
          pltpu.SemaphoreType.DMA,
          pltpu.SemaphoreType.DMA,
          pltpu.SemaphoreType.DMA,
      ],
  )
  def k(up_hbm, idx_hbm, plt_hbm, out_hbm, idx_v, idxc_v, valf_v, val_v,
        cb_a, cb_b, sem, lsem, ssem):
    n = lax.axis_index("c")
    sid = lax.axis_index("s")
    bufs = [cb_a, cb_b]
    st = [None, None]

    def cbase(c):
      return (n * NCLS + c) * hw + sid * cpix

    for c in range(NCLS):
      b = c % 2
      if st[b] is not None:
        st[b].wait()
      pltpu.async_copy(up_hbm.at[pl.ds(cbase(c), cpix)], bufs[b], lsem).wait()
      st[b] = pltpu.async_copy(bufs[b], out_hbm.at[pl.ds(cbase(c), cpix)],
                               ssem)
    st[0].wait()
    st[1].wait()
    pltpu.sync_copy(idx_hbm.at[pl.ds(n * IDXPAD + sid * ppt, ppt)], idx_v)
    plsc.subcore_barrier()
    inflight = [None, None]
    for c in range(NCLS):
      b = c % 2
      if inflight[b] is not None:
        for cp in inflight[b]:
          cp.wait()
      pltpu.sync_copy(
          plt_hbm.at[pl.ds((n * 32 + c) * NPT + sid * ppt, ppt)],
          valf_v.at[b])
      chb = jnp.full((16,), (n * NCLS + c) * hw, jnp.int32)
      for j in range(4):
        for kk in range(8):
          s = pl.ds(j * 128 + kk * 16, 16)
          s2 = pl.ds(kk * 16, 16)
          idxc_v[b, j, s2] = idx_v[s] + chb
          val_v[b, j, s2] = valf_v[b, s]
      inflight[b] = [
          pltpu.async_copy(val_v.at[b, j], out_hbm.at[idxc_v.at[b, j]], sem)
          for j in range(4)
      ]
    for h in inflight:
      if h is not None:
        for cp in h:
          cp.wait()

  return k


def kernel(features, coarse_logits, W1, b1, W2, b2, W3, b3, Wp, bp):
  n, f, h0, w0 = features.shape
  c = coarse_logits.shape[1]

  tab = jnp.concatenate([features, coarse_logits], axis=1)
  tab = tab.reshape(n, f + c, h0 * w0).transpose(0, 2, 1)
  tab = jnp.pad(tab, ((0, 0), (0, 0), (0, 128 - (f + c))))

  w1t = jnp.zeros((128, FDIM), jnp.float32).at[: f + c].set(W1.T)
  w2h = W2[:, :FDIM].T
  w2c = jnp.zeros((128, FDIM), jnp.float32).at[f : f + c].set(W2[:, FDIM:].T)
  w3h = W3[:, :FDIM].T
  w3c = jnp.zeros((128, FDIM), jnp.float32).at[f : f + c].set(W3[:, FDIM:].T)
  wph = jnp.zeros((FDIM, 32), jnp.float32).at[:, :c].set(Wp[:, :FDIM].T)
  wpc = jnp.zeros((128, 32), jnp.float32).at[f : f + c, :c].set(Wp[:, FDIM:].T)
  b1r = jnp.broadcast_to(b1, (8, FDIM))
  b2r = jnp.broadcast_to(b2, (8, FDIM))
  b3r = jnp.broadcast_to(b3, (8, FDIM))
  bpr = jnp.zeros((8, 32), jnp.float32).at[0, :c].set(bp)
  wd = [w1t, w2h, w2c, w3h, w3c, wph, wpc, b1r, b2r, b3r, bpr]

  sem = coarse_logits
  for step in range(2):
    hin = sem.shape[2]
    h2 = 2 * hin
    hw = h2 * h2
    uw = _upmat(hin)
    uh = uw.T
    xw = _k1a(sem, uw)
    up, key = _k1b(xw, uh)
    idx = _make_topk(hw)(key.reshape(n * hw))
    shift = 1 if step == 0 else 2
    wtab = (0.75, 0.25) if step == 0 else (0.625, 0.875, 0.125, 0.375)
    pts = _make_gather(h2, shift, wtab)(idx, tab)
    po = _k4(pts, wd)
    plt = jnp.swapaxes(po, 1, 2).reshape(n * 32 * NPT)
    out = _make_scatter(hw)(up.reshape(n * c * hw), idx, plt)
    sem = out.reshape(n, c, h2, h2)
  return sem
